# baseline (speedup 1.0000x reference)
import functools
import jax
import jax.numpy as jnp
from jax import lax
from jax.experimental import pallas as pl
from jax.experimental.pallas import tpu as pltpu
from jax.experimental.pallas import tpu_sc as plsc

T = 32768
D = 768
E = 64
TB = 2048

NC = 2
NS = 16
NW = NC * NS
TPW = T // NW
C = 512


def _matmul_body(x_ref, sim_ref, logits_ref):
    x = x_ref[...]
    sim = sim_ref[...]
    sn = sim / jnp.clip(
        jnp.sqrt(jnp.sum(sim * sim, axis=0, keepdims=True)), 1e-12
    )
    xn = x * (1.0 / jnp.maximum(
        jnp.sqrt(jnp.sum(x * x, axis=1, keepdims=True)), 1e-12))
    logits_ref[...] = jnp.dot(xn, sn, preferred_element_type=jnp.float32)


def _tc_logits(x, sim_matrix):
    return pl.pallas_call(
        _matmul_body,
        grid=(T // TB,),
        in_specs=[
            pl.BlockSpec((TB, D), lambda i: (i, 0)),
            pl.BlockSpec((D, E), lambda i: (0, 0)),
        ],
        out_specs=pl.BlockSpec((TB, E), lambda i: (i, 0)),
        out_shape=jax.ShapeDtypeStruct((T, E), jnp.float32),
        compiler_params=pltpu.CompilerParams(
            dimension_semantics=("arbitrary",),
        ),
    )(x, sim_matrix)


def _routing_body(logits_hbm, gates_hbm, mask_hbm, probs_hbm,
                  g_v, lg_v, mk_v, pb_v):
    wid = lax.axis_index("s") * NC + lax.axis_index("c")
    base = wid * TPW

    pltpu.sync_copy(gates_hbm, g_v)
    thr = []
    ids = []
    for j in range(4):
        g = g_v[pl.ds(j * 16, 16)]
        thr.append(1.0 / (1.0 + jnp.exp(-g)))
        ids.append(lax.broadcasted_iota(jnp.int32, (16,), 0) + (j * 16))

    for c in range(TPW // C):
        tok = base + c * C
        pltpu.sync_copy(logits_hbm.at[pl.ds(tok, C), :], lg_v)

        def body(t, carry):
            l = [lg_v[t, pl.ds(j * 16, 16)] for j in range(4)]
            gated = [jnp.maximum(l[j] - thr[j], 0.0) for j in range(4)]
            mask = [jnp.sign(gated[j]) for j in range(4)]
            cnt = jnp.sum(mask[0] + mask[1] + mask[2] + mask[3])
            mx = jnp.max(jnp.maximum(jnp.maximum(l[0], l[1]),
                                     jnp.maximum(l[2], l[3])))
            idx = [jnp.where(l[j] == mx, ids[j], jnp.int32(E))
                   for j in range(4)]
            top1 = jnp.min(jnp.minimum(jnp.minimum(idx[0], idx[1]),
                                       jnp.minimum(idx[2], idx[3])))
            inactive = cnt == 0.0
            mask = [jnp.where(jnp.logical_and(inactive, ids[j] == top1),
                              1.0, mask[j]) for j in range(4)]
            gm = [jnp.where(mask[j] > 0.0, gated[j], jnp.float32(-1e9))
                  for j in range(4)]
            m2 = jnp.max(jnp.maximum(jnp.maximum(gm[0], gm[1]),
                                     jnp.maximum(gm[2], gm[3])))
            ex = [jnp.exp(gm[j] - m2) for j in range(4)]
            s = jnp.sum(ex[0] + ex[1] + ex[2] + ex[3])
            invv = 1.0 / (jnp.zeros((16,), jnp.float32) + s)
            for j in range(4):
                mk_v[t, pl.ds(j * 16, 16)] = mask[j]
                pb_v[t, pl.ds(j * 16, 16)] = ex[j] * invv
            return carry

        lax.fori_loop(0, C, body, 0, unroll=2)

        pltpu.sync_copy(mk_v, mask_hbm.at[pl.ds(tok, C), :])
        pltpu.sync_copy(pb_v, probs_hbm.at[pl.ds(tok, C), :])


def _sc_routing(logits, gates):
    mesh = plsc.VectorSubcoreMesh(
        core_axis_name="c", subcore_axis_name="s",
        num_cores=NC, num_subcores=NS)
    fn = functools.partial(
        pl.kernel,
        mesh=mesh,
        compiler_params=pltpu.CompilerParams(
            use_tc_tiling_on_sc=False, needs_layout_passes=False),
        out_type=(
            jax.ShapeDtypeStruct((T, E), jnp.float32),
            jax.ShapeDtypeStruct((T, E), jnp.float32),
        ),
        scratch_types=[
            pltpu.VMEM((E,), jnp.float32),
            pltpu.VMEM((C, E), jnp.float32),
            pltpu.VMEM((C, E), jnp.float32),
            pltpu.VMEM((C, E), jnp.float32),
        ],
    )(_routing_body)
    return fn(logits, gates)


def kernel(x, sim_matrix, gates):
    logits = _tc_logits(x, sim_matrix)
    mask, probs = _sc_routing(logits, gates)
    return (mask, probs, logits)


# speedup vs baseline: 1.0417x; 1.0417x over previous
import functools
import jax
import jax.numpy as jnp
from jax import lax
from jax.experimental import pallas as pl
from jax.experimental.pallas import tpu as pltpu
from jax.experimental.pallas import tpu_sc as plsc

T = 32768
D = 768
E = 64
TB = 2048

NC = 2
NS = 16
NW = NC * NS
TPW = T // NW
C = 512


def _matmul_body(x_ref, sim_ref, logits_ref):
    x = x_ref[...]
    sim = sim_ref[...]
    sn = sim / jnp.clip(
        jnp.sqrt(jnp.sum(sim * sim, axis=0, keepdims=True)), 1e-12
    )
    xn = x * (1.0 / jnp.maximum(
        jnp.sqrt(jnp.sum(x * x, axis=1, keepdims=True)), 1e-12))
    logits_ref[...] = jnp.dot(xn, sn, preferred_element_type=jnp.float32)


def _tc_logits(x, sim_matrix):
    return pl.pallas_call(
        _matmul_body,
        grid=(T // TB,),
        in_specs=[
            pl.BlockSpec((TB, D), lambda i: (i, 0)),
            pl.BlockSpec((D, E), lambda i: (0, 0)),
        ],
        out_specs=pl.BlockSpec((TB, E), lambda i: (i, 0)),
        out_shape=jax.ShapeDtypeStruct((T, E), jnp.float32),
        compiler_params=pltpu.CompilerParams(
            dimension_semantics=("arbitrary",),
        ),
    )(x, sim_matrix)


def _routing_body(logits_hbm, gates_hbm, mask_hbm, probs_hbm,
                  g_v, lg_v, mk_v, pb_v):
    wid = lax.axis_index("s") * NC + lax.axis_index("c")
    base = wid * TPW

    pltpu.sync_copy(gates_hbm, g_v)
    thr = []
    ids = []
    for j in range(4):
        g = g_v[pl.ds(j * 16, 16)]
        thr.append(1.0 / (1.0 + jnp.exp(-g)))
        ids.append(lax.broadcasted_iota(jnp.int32, (16,), 0) + (j * 16))

    for c in range(TPW // C):
        tok = base + c * C
        pltpu.sync_copy(logits_hbm.at[pl.ds(tok, C), :], lg_v)

        def body(t, carry):
            l = [lg_v[t, pl.ds(j * 16, 16)] for j in range(4)]
            gated = [jnp.maximum(l[j] - thr[j], 0.0) for j in range(4)]
            mask = [jnp.sign(gated[j]) for j in range(4)]
            gmax = jnp.max(jnp.maximum(jnp.maximum(gated[0], gated[1]),
                                       jnp.maximum(gated[2], gated[3])))
            mx = jnp.max(jnp.maximum(jnp.maximum(l[0], l[1]),
                                     jnp.maximum(l[2], l[3])))
            idx = [jnp.where(l[j] == mx, ids[j], jnp.int32(E))
                   for j in range(4)]
            top1 = jnp.min(jnp.minimum(jnp.minimum(idx[0], idx[1]),
                                       jnp.minimum(idx[2], idx[3])))
            inactive = gmax == 0.0
            mask = [jnp.where(jnp.logical_and(inactive, ids[j] == top1),
                              1.0, mask[j]) for j in range(4)]
            gm = [jnp.where(mask[j] > 0.0, gated[j], jnp.float32(-1e9))
                  for j in range(4)]
            ex = [jnp.exp(gm[j] - gmax) for j in range(4)]
            s = jnp.sum(ex[0] + ex[1] + ex[2] + ex[3])
            invv = 1.0 / (jnp.zeros((16,), jnp.float32) + s)
            for j in range(4):
                mk_v[t, pl.ds(j * 16, 16)] = mask[j]
                pb_v[t, pl.ds(j * 16, 16)] = ex[j] * invv
            return carry

        lax.fori_loop(0, C, body, 0, unroll=4)

        pltpu.sync_copy(mk_v, mask_hbm.at[pl.ds(tok, C), :])
        pltpu.sync_copy(pb_v, probs_hbm.at[pl.ds(tok, C), :])


def _sc_routing(logits, gates):
    mesh = plsc.VectorSubcoreMesh(
        core_axis_name="c", subcore_axis_name="s",
        num_cores=NC, num_subcores=NS)
    fn = functools.partial(
        pl.kernel,
        mesh=mesh,
        compiler_params=pltpu.CompilerParams(
            use_tc_tiling_on_sc=False, needs_layout_passes=False),
        out_type=(
            jax.ShapeDtypeStruct((T, E), jnp.float32),
            jax.ShapeDtypeStruct((T, E), jnp.float32),
        ),
        scratch_types=[
            pltpu.VMEM((E,), jnp.float32),
            pltpu.VMEM((C, E), jnp.float32),
            pltpu.VMEM((C, E), jnp.float32),
            pltpu.VMEM((C, E), jnp.float32),
        ],
    )(_routing_body)
    return fn(logits, gates)


def kernel(x, sim_matrix, gates):
    logits = _tc_logits(x, sim_matrix)
    mask, probs = _sc_routing(logits, gates)
    return (mask, probs, logits)


# speedup vs baseline: 1.1538x; 1.1076x over previous
import functools
import jax
import jax.numpy as jnp
from jax import lax
from jax.experimental import pallas as pl
from jax.experimental.pallas import tpu as pltpu
from jax.experimental.pallas import tpu_sc as plsc

T = 32768
D = 768
E = 64
TB = 2048

NC = 2
NS = 16
NW = NC * NS
TPW = T // NW
C = 128


def _matmul_body(x_ref, sim_ref, logits_ref):
    x = x_ref[...]
    sim = sim_ref[...]
    sn = sim / jnp.clip(
        jnp.sqrt(jnp.sum(sim * sim, axis=0, keepdims=True)), 1e-12
    )
    xn = x * (1.0 / jnp.maximum(
        jnp.sqrt(jnp.sum(x * x, axis=1, keepdims=True)), 1e-12))
    logits_ref[...] = jnp.dot(xn, sn, preferred_element_type=jnp.float32)


def _tc_logits(x, sim_matrix):
    return pl.pallas_call(
        _matmul_body,
        grid=(T // TB,),
        in_specs=[
            pl.BlockSpec((TB, D), lambda i: (i, 0)),
            pl.BlockSpec((D, E), lambda i: (0, 0)),
        ],
        out_specs=pl.BlockSpec((TB, E), lambda i: (i, 0)),
        out_shape=jax.ShapeDtypeStruct((T, E), jnp.float32),
        compiler_params=pltpu.CompilerParams(
            dimension_semantics=("arbitrary",),
        ),
    )(x, sim_matrix)


def _routing_body(logits_hbm, gates_hbm, mask_hbm, probs_hbm,
                  g_v, lg_v, mk_v, pb_v):
    wid = lax.axis_index("s") * NC + lax.axis_index("c")
    base = wid * TPW

    pltpu.sync_copy(gates_hbm, g_v)
    thr = []
    ids = []
    for j in range(4):
        g = g_v[pl.ds(j * 16, 16)]
        thr.append(1.0 / (1.0 + jnp.exp(-g)))
        ids.append(lax.broadcasted_iota(jnp.int32, (16,), 0) + (j * 16))

    for c in range(TPW // C):
        tok = base + c * C
        pltpu.sync_copy(logits_hbm.at[pl.ds(tok, C), :], lg_v)

        def body(t, carry):
            l = [lg_v[t, pl.ds(j * 16, 16)] for j in range(4)]
            gated = [jnp.maximum(l[j] - thr[j], 0.0) for j in range(4)]
            mask = [jnp.sign(gated[j]) for j in range(4)]
            gmax = jnp.max(jnp.maximum(jnp.maximum(gated[0], gated[1]),
                                       jnp.maximum(gated[2], gated[3])))
            mx = jnp.max(jnp.maximum(jnp.maximum(l[0], l[1]),
                                     jnp.maximum(l[2], l[3])))
            idx = [jnp.where(l[j] == mx, ids[j], jnp.int32(E))
                   for j in range(4)]
            top1 = jnp.min(jnp.minimum(jnp.minimum(idx[0], idx[1]),
                                       jnp.minimum(idx[2], idx[3])))
            inactive = gmax == 0.0
            mask = [jnp.where(jnp.logical_and(inactive, ids[j] == top1),
                              1.0, mask[j]) for j in range(4)]
            gm = [jnp.where(mask[j] > 0.0, gated[j], jnp.float32(-1e9))
                  for j in range(4)]
            ex = [jnp.exp(gm[j] - gmax) for j in range(4)]
            s = jnp.sum(ex[0] + ex[1] + ex[2] + ex[3])
            invv = 1.0 / (jnp.zeros((16,), jnp.float32) + s)
            for j in range(4):
                mk_v[t, pl.ds(j * 16, 16)] = mask[j]
                pb_v[t, pl.ds(j * 16, 16)] = ex[j] * invv
            return carry

        lax.fori_loop(0, C, body, 0, unroll=4)

        pltpu.sync_copy(mk_v, mask_hbm.at[pl.ds(tok, C), :])
        pltpu.sync_copy(pb_v, probs_hbm.at[pl.ds(tok, C), :])


def _sc_routing(logits, gates):
    mesh = plsc.VectorSubcoreMesh(
        core_axis_name="c", subcore_axis_name="s",
        num_cores=NC, num_subcores=NS)
    fn = functools.partial(
        pl.kernel,
        mesh=mesh,
        compiler_params=pltpu.CompilerParams(
            use_tc_tiling_on_sc=True, needs_layout_passes=False),
        out_type=(
            jax.ShapeDtypeStruct((T, E), jnp.float32),
            jax.ShapeDtypeStruct((T, E), jnp.float32),
        ),
        scratch_types=[
            pltpu.VMEM((E,), jnp.float32),
            pltpu.VMEM((C, E), jnp.float32),
            pltpu.VMEM((C, E), jnp.float32),
            pltpu.VMEM((C, E), jnp.float32),
        ],
    )(_routing_body)
    return fn(logits, gates)


def kernel(x, sim_matrix, gates):
    logits = _tc_logits(x, sim_matrix)
    mask, probs = _sc_routing(logits, gates)
    return (mask, probs, logits)


# speedup vs baseline: 1.3482x; 1.1685x over previous
import functools
import jax
import jax.numpy as jnp
from jax import lax
from jax.experimental import pallas as pl
from jax.experimental.pallas import tpu as pltpu
from jax.experimental.pallas import tpu_sc as plsc

T = 32768
D = 768
E = 64
TB = 2048

NC = 2
NS = 16
NW = NC * NS
TPW = T // NW
C = 128


def _matmul_body(x_ref, sim_ref, logits_ref):
    x = x_ref[...]
    sim = sim_ref[...]
    sn = sim / jnp.clip(
        jnp.sqrt(jnp.sum(sim * sim, axis=0, keepdims=True)), 1e-12
    )
    xn = x * (1.0 / jnp.maximum(
        jnp.sqrt(jnp.sum(x * x, axis=1, keepdims=True)), 1e-12))
    logits_ref[...] = jnp.dot(xn, sn, preferred_element_type=jnp.float32)


def _tc_logits(x, sim_matrix):
    return pl.pallas_call(
        _matmul_body,
        grid=(T // TB,),
        in_specs=[
            pl.BlockSpec((TB, D), lambda i: (i, 0)),
            pl.BlockSpec((D, E), lambda i: (0, 0)),
        ],
        out_specs=pl.BlockSpec((TB, E), lambda i: (i, 0)),
        out_shape=jax.ShapeDtypeStruct((T, E), jnp.float32),
        compiler_params=pltpu.CompilerParams(
            dimension_semantics=("arbitrary",),
        ),
    )(x, sim_matrix)


def _routing_body(logits_hbm, gates_hbm, mask_hbm, probs_hbm,
                  g_v, lg_v, mk_v, pb_v):
    wid = lax.axis_index("s") * NC + lax.axis_index("c")
    base = wid * TPW

    pltpu.sync_copy(gates_hbm, g_v)
    thr = []
    ids = []
    for j in range(4):
        g = g_v[pl.ds(j * 16, 16)]
        thr.append(1.0 / (1.0 + jnp.exp(-g)))
        ids.append(lax.broadcasted_iota(jnp.int32, (16,), 0) + (j * 16))

    for c in range(TPW // C):
        tok = base + c * C
        pltpu.sync_copy(logits_hbm.at[pl.ds(tok, C), :], lg_v)

        def body(t):
            l = [lg_v[t, pl.ds(j * 16, 16)] for j in range(4)]
            gated = [jnp.maximum(l[j] - thr[j], 0.0) for j in range(4)]
            mask = [jnp.sign(gated[j]) for j in range(4)]
            gmax = jnp.max(jnp.maximum(jnp.maximum(gated[0], gated[1]),
                                       jnp.maximum(gated[2], gated[3])))
            mx = jnp.max(jnp.maximum(jnp.maximum(l[0], l[1]),
                                     jnp.maximum(l[2], l[3])))
            idx = [jnp.where(l[j] == mx, ids[j], jnp.int32(E))
                   for j in range(4)]
            top1 = jnp.min(jnp.minimum(jnp.minimum(idx[0], idx[1]),
                                       jnp.minimum(idx[2], idx[3])))
            inactive = gmax == 0.0
            mask = [jnp.where(jnp.logical_and(inactive, ids[j] == top1),
                              1.0, mask[j]) for j in range(4)]
            gm = [jnp.where(mask[j] > 0.0, gated[j], jnp.float32(-1e9))
                  for j in range(4)]
            ex = [jnp.exp(gm[j] - gmax) for j in range(4)]
            s = jnp.sum(ex[0] + ex[1] + ex[2] + ex[3])
            invv = 1.0 / (jnp.zeros((16,), jnp.float32) + s)
            for j in range(4):
                mk_v[t, pl.ds(j * 16, 16)] = mask[j]
                pb_v[t, pl.ds(j * 16, 16)] = ex[j] * invv

        plsc.parallel_loop(0, C, 1, unroll=8)(body)

        pltpu.sync_copy(mk_v, mask_hbm.at[pl.ds(tok, C), :])
        pltpu.sync_copy(pb_v, probs_hbm.at[pl.ds(tok, C), :])


def _sc_routing(logits, gates):
    mesh = plsc.VectorSubcoreMesh(
        core_axis_name="c", subcore_axis_name="s",
        num_cores=NC, num_subcores=NS)
    fn = functools.partial(
        pl.kernel,
        mesh=mesh,
        compiler_params=pltpu.CompilerParams(
            use_tc_tiling_on_sc=True, needs_layout_passes=False),
        out_type=(
            jax.ShapeDtypeStruct((T, E), jnp.float32),
            jax.ShapeDtypeStruct((T, E), jnp.float32),
        ),
        scratch_types=[
            pltpu.VMEM((E,), jnp.float32),
            pltpu.VMEM((C, E), jnp.float32),
            pltpu.VMEM((C, E), jnp.float32),
            pltpu.VMEM((C, E), jnp.float32),
        ],
    )(_routing_body)
    return fn(logits, gates)


def kernel(x, sim_matrix, gates):
    logits = _tc_logits(x, sim_matrix)
    mask, probs = _sc_routing(logits, gates)
    return (mask, probs, logits)
